# Initial kernel scaffold; baseline (speedup 1.0000x reference)
import jax
import jax.numpy as jnp
from jax.experimental import pallas as pl


def kernel(x, edge_index, W1, b1, W2, b2):
    raise NotImplementedError("write your pallas kernel here")


# speedup vs baseline: 5.1392x; 5.1392x over previous
import functools

import jax
import jax.numpy as jnp
from jax import lax
from jax.experimental import pallas as pl
from jax.experimental.pallas import tpu as pltpu
from jax.experimental.pallas import tpu_sc as plsc

N_NODES = 10000
N_EDGES = 160000
D = 256
DH = D // 2
NC = 2
NS = 16
EDGES_PER_TILE = N_EDGES // NS
CHUNK = 80
NCHUNK = EDGES_PER_TILE // CHUNK
ROWS_PER_TILE = 624
TAIL_ROWS = N_NODES - NS * ROWS_PER_TILE
TAIL_OFF = NS * ROWS_PER_TILE


def _sc_body(x_lo, x_hi, src3, dst3, zeros, out_lo, out_hi,
             src_vm, dst_vm, rows_v, agg_sh, sem):
    c = lax.axis_index("c")
    s = lax.axis_index("s")

    r0 = s * ROWS_PER_TILE
    pltpu.sync_copy(zeros.at[pl.ds(r0, ROWS_PER_TILE)],
                    agg_sh.at[pl.ds(r0, ROWS_PER_TILE)])
    pl.when(s == 0)(lambda: pltpu.sync_copy(
        zeros.at[pl.ds(TAIL_OFF, TAIL_ROWS)],
        agg_sh.at[pl.ds(TAIL_OFF, TAIL_ROWS)]))
    pltpu.sync_copy(src3.at[s], src_vm)
    pltpu.sync_copy(dst3.at[s], dst_vm)
    plsc.subcore_barrier()

    def do_half(table):
        def chunk(j, carry):
            pltpu.async_copy(table.at[src_vm.at[j]], rows_v, sem).wait()
            pltpu.sync_copy(rows_v, agg_sh.at[dst_vm.at[j]], add=True)
            return carry
        lax.fori_loop(0, NCHUNK, chunk, 0)

    pl.when(c == 0)(lambda: do_half(x_lo))
    pl.when(c == 1)(lambda: do_half(x_hi))
    plsc.subcore_barrier()

    def writeout(out_ref):
        pltpu.sync_copy(agg_sh.at[pl.ds(r0, ROWS_PER_TILE)],
                        out_ref.at[pl.ds(r0, ROWS_PER_TILE)])
        pl.when(s == 0)(lambda: pltpu.sync_copy(
            agg_sh.at[pl.ds(TAIL_OFF, TAIL_ROWS)],
            out_ref.at[pl.ds(TAIL_OFF, TAIL_ROWS)]))

    pl.when(c == 0)(lambda: writeout(out_lo))
    pl.when(c == 1)(lambda: writeout(out_hi))


_sc_scatter = functools.partial(
    pl.kernel,
    out_type=(jax.ShapeDtypeStruct((N_NODES, DH), jnp.float32),
              jax.ShapeDtypeStruct((N_NODES, DH), jnp.float32)),
    mesh=plsc.VectorSubcoreMesh(core_axis_name="c", subcore_axis_name="s",
                                num_cores=NC, num_subcores=NS),
    scratch_types=[
        pltpu.VMEM((NCHUNK, CHUNK), jnp.int32),
        pltpu.VMEM((NCHUNK, CHUNK), jnp.int32),
        pltpu.VMEM((CHUNK, DH), jnp.float32),
        pltpu.VMEM_SHARED((N_NODES, DH), jnp.float32),
        pltpu.SemaphoreType.DMA,
    ],
)(_sc_body)


def _mlp_body(x_ref, lo_ref, hi_ref, w1_ref, b1_ref, w2_ref, b2_ref, o_ref):
    h = x_ref[...] + jnp.concatenate([lo_ref[...], hi_ref[...]], axis=1)
    h = jnp.dot(h, w1_ref[...], preferred_element_type=jnp.float32) + b1_ref[...]
    h = jnp.maximum(h, 0.0)
    o_ref[...] = (jnp.dot(h, w2_ref[...], preferred_element_type=jnp.float32)
                  + b2_ref[...])


BLK = 1000


def _mlp(x, agg_lo, agg_hi, w1, b1, w2, b2):
    return pl.pallas_call(
        _mlp_body,
        grid=(N_NODES // BLK,),
        in_specs=[
            pl.BlockSpec((BLK, D), lambda i: (i, 0)),
            pl.BlockSpec((BLK, DH), lambda i: (i, 0)),
            pl.BlockSpec((BLK, DH), lambda i: (i, 0)),
            pl.BlockSpec((D, D), lambda i: (0, 0)),
            pl.BlockSpec((1, D), lambda i: (0, 0)),
            pl.BlockSpec((D, D), lambda i: (0, 0)),
            pl.BlockSpec((1, D), lambda i: (0, 0)),
        ],
        out_specs=pl.BlockSpec((BLK, D), lambda i: (i, 0)),
        out_shape=jax.ShapeDtypeStruct((N_NODES, D), jnp.float32),
    )(x, agg_lo, agg_hi, w1, b1.reshape(1, D), w2, b2.reshape(1, D))


def kernel(x, edge_index, W1, b1, W2, b2):
    src = edge_index[0].astype(jnp.int32).reshape(NS, NCHUNK, CHUNK)
    dst = edge_index[1].astype(jnp.int32).reshape(NS, NCHUNK, CHUNK)
    x_lo = x[:, :DH]
    x_hi = x[:, DH:]
    zeros = jnp.zeros((N_NODES, DH), jnp.float32)
    agg_lo, agg_hi = _sc_scatter(x_lo, x_hi, src, dst, zeros)
    return _mlp(x, agg_lo, agg_hi, W1, b1, W2, b2)


# speedup vs baseline: 8.1562x; 1.5870x over previous
import functools

import jax
import jax.numpy as jnp
from jax import lax
from jax.experimental import pallas as pl
from jax.experimental.pallas import tpu as pltpu
from jax.experimental.pallas import tpu_sc as plsc

N_NODES = 10000
N_EDGES = 160000
D = 256
DQ = D // 2
NC = 2
NS = 16
EDGES_PER_TILE = N_EDGES // NS
CHUNK = 125
NCHUNK = EDGES_PER_TILE // CHUNK
HALF = NCHUNK // 2
ROWS_PER_TILE = 624
TAIL_ROWS = N_NODES - NS * ROWS_PER_TILE
TAIL_OFF = NS * ROWS_PER_TILE


def _sc_body(x0, x1, srcdst, zeros, out0, out1,
             sd_vm, rows0, rows1, agg_sh, gsem0, gsem1):
    c = lax.axis_index("c")
    s = lax.axis_index("s")
    r0 = s * ROWS_PER_TILE

    def zero_agg():
        pltpu.sync_copy(zeros.at[pl.ds(r0, ROWS_PER_TILE)],
                        agg_sh.at[pl.ds(r0, ROWS_PER_TILE)])
        pl.when(s == 0)(lambda: pltpu.sync_copy(
            zeros.at[pl.ds(TAIL_OFF, TAIL_ROWS)],
            agg_sh.at[pl.ds(TAIL_OFF, TAIL_ROWS)]))

    def accumulate(table):
        bufs = ((rows0, gsem0), (rows1, gsem1))
        for h in range(NCHUNK // HALF):
            pltpu.sync_copy(srcdst.at[s, pl.ds(h * HALF, HALF)], sd_vm)
            for b, (buf, gsem) in enumerate(bufs):
                pltpu.async_copy(table.at[sd_vm.at[b, 0]], buf, gsem)

            def outer(j, carry):
                for b, (buf, gsem) in enumerate(bufs):
                    jj = 2 * j + b
                    pltpu.make_async_copy(
                        table.at[sd_vm.at[jj, 0]], buf, gsem).wait()
                    pltpu.sync_copy(buf, agg_sh.at[sd_vm.at[jj, 1]], add=True)
                    @pl.when(jj + 2 < HALF)
                    def _(buf=buf, gsem=gsem, jj=jj):
                        pltpu.async_copy(table.at[sd_vm.at[jj + 2, 0]], buf, gsem)
                return carry
            lax.fori_loop(0, HALF // 2, outer, 0)

    def writeout(out_ref):
        pltpu.sync_copy(agg_sh.at[pl.ds(r0, ROWS_PER_TILE)],
                        out_ref.at[pl.ds(r0, ROWS_PER_TILE)])
        pl.when(s == 0)(lambda: pltpu.sync_copy(
            agg_sh.at[pl.ds(TAIL_OFF, TAIL_ROWS)],
            out_ref.at[pl.ds(TAIL_OFF, TAIL_ROWS)]))

    zero_agg()
    plsc.subcore_barrier()

    pl.when(c == 0)(lambda: accumulate(x0))
    pl.when(c == 1)(lambda: accumulate(x1))
    plsc.subcore_barrier()
    pl.when(c == 0)(lambda: writeout(out0))
    pl.when(c == 1)(lambda: writeout(out1))


_quarter = jax.ShapeDtypeStruct((N_NODES, DQ), jnp.float32)
_sc_scatter = functools.partial(
    pl.kernel,
    out_type=(_quarter, _quarter),
    mesh=plsc.VectorSubcoreMesh(core_axis_name="c", subcore_axis_name="s",
                                num_cores=NC, num_subcores=NS),
    scratch_types=[
        pltpu.VMEM((HALF, 2, CHUNK), jnp.int32),
        pltpu.VMEM((CHUNK, DQ), jnp.float32),
        pltpu.VMEM((CHUNK, DQ), jnp.float32),
        pltpu.VMEM_SHARED((N_NODES, DQ), jnp.float32),
        pltpu.SemaphoreType.DMA,
        pltpu.SemaphoreType.DMA,
    ],
)(_sc_body)


def _mlp_body(x_ref, a0_ref, a1_ref,
              w1_ref, b1_ref, w2_ref, b2_ref, o_ref):
    h = x_ref[...] + jnp.concatenate([a0_ref[...], a1_ref[...]], axis=1)
    h = jnp.dot(h, w1_ref[...], preferred_element_type=jnp.float32) + b1_ref[...]
    h = jnp.maximum(h, 0.0)
    o_ref[...] = (jnp.dot(h, w2_ref[...], preferred_element_type=jnp.float32)
                  + b2_ref[...])


BLK = 1000


def _mlp(x, aggs, w1, b1, w2, b2):
    return pl.pallas_call(
        _mlp_body,
        grid=(N_NODES // BLK,),
        in_specs=[
            pl.BlockSpec((BLK, D), lambda i: (i, 0)),
            pl.BlockSpec((BLK, DQ), lambda i: (i, 0)),
            pl.BlockSpec((BLK, DQ), lambda i: (i, 0)),
            pl.BlockSpec((D, D), lambda i: (0, 0)),
            pl.BlockSpec((1, D), lambda i: (0, 0)),
            pl.BlockSpec((D, D), lambda i: (0, 0)),
            pl.BlockSpec((1, D), lambda i: (0, 0)),
        ],
        out_specs=pl.BlockSpec((BLK, D), lambda i: (i, 0)),
        out_shape=jax.ShapeDtypeStruct((N_NODES, D), jnp.float32),
    )(x, *aggs, w1, b1.reshape(1, D), w2, b2.reshape(1, D))


def kernel(x, edge_index, W1, b1, W2, b2):
    src = edge_index[0].astype(jnp.int32).reshape(NS, NCHUNK, CHUNK)
    dst = edge_index[1].astype(jnp.int32).reshape(NS, NCHUNK, CHUNK)
    srcdst = jnp.stack([src, dst], axis=2)
    xq = [x[:, q * DQ:(q + 1) * DQ] for q in range(2)]
    zeros = jnp.zeros((N_NODES, DQ), jnp.float32)
    aggs = _sc_scatter(*xq, srcdst, zeros)
    return _mlp(x, aggs, W1, b1, W2, b2)


# speedup vs baseline: 8.2664x; 1.0135x over previous
import functools

import jax
import jax.numpy as jnp
from jax import lax
from jax.experimental import pallas as pl
from jax.experimental.pallas import tpu as pltpu
from jax.experimental.pallas import tpu_sc as plsc

N_NODES = 10000
N_EDGES = 160000
D = 256
DQ = D // 2
NC = 2
NS = 16
EDGES_PER_TILE = N_EDGES // NS
CHUNK = 125
NCHUNK = EDGES_PER_TILE // CHUNK
HALF = NCHUNK // 2
ROWS_PER_TILE = 624
TAIL_ROWS = N_NODES - NS * ROWS_PER_TILE
TAIL_OFF = NS * ROWS_PER_TILE


def _sc_body(x_all, src4, dst4, zeros, out0, out1,
             src_vm, dst_vm, rows0, rows1, agg_sh, gsem0, gsem1):
    c = lax.axis_index("c")
    s = lax.axis_index("s")
    r0 = s * ROWS_PER_TILE

    def zero_agg():
        pltpu.sync_copy(zeros, agg_sh.at[pl.ds(r0, ROWS_PER_TILE)])
        pl.when(s == 0)(lambda: pltpu.sync_copy(
            zeros.at[pl.ds(0, TAIL_ROWS)],
            agg_sh.at[pl.ds(TAIL_OFF, TAIL_ROWS)]))

    def accumulate(col0):
        table = x_all.at[:, pl.ds(col0, DQ)]
        bufs = ((rows0, gsem0), (rows1, gsem1))
        for h in range(NCHUNK // HALF):
            pltpu.sync_copy(src4.at[s, pl.ds(h * HALF, HALF)], src_vm)
            pltpu.sync_copy(dst4.at[s, pl.ds(h * HALF, HALF)], dst_vm)
            for b, (buf, gsem) in enumerate(bufs):
                pltpu.async_copy(table.at[src_vm.at[b]], buf, gsem)

            def outer(j, carry):
                for b, (buf, gsem) in enumerate(bufs):
                    jj = 2 * j + b
                    pltpu.make_async_copy(
                        table.at[src_vm.at[jj]], buf, gsem).wait()
                    pltpu.sync_copy(buf, agg_sh.at[dst_vm.at[jj]], add=True)
                    @pl.when(jj + 2 < HALF)
                    def _(buf=buf, gsem=gsem, jj=jj):
                        pltpu.async_copy(table.at[src_vm.at[jj + 2]], buf, gsem)
                return carry
            lax.fori_loop(0, HALF // 2, outer, 0)

    def writeout(out_ref):
        pltpu.sync_copy(agg_sh.at[pl.ds(r0, ROWS_PER_TILE)],
                        out_ref.at[pl.ds(r0, ROWS_PER_TILE)])
        pl.when(s == 0)(lambda: pltpu.sync_copy(
            agg_sh.at[pl.ds(TAIL_OFF, TAIL_ROWS)],
            out_ref.at[pl.ds(TAIL_OFF, TAIL_ROWS)]))

    zero_agg()
    plsc.subcore_barrier()

    accumulate(c * DQ)
    plsc.subcore_barrier()
    pl.when(c == 0)(lambda: writeout(out0))
    pl.when(c == 1)(lambda: writeout(out1))


_quarter = jax.ShapeDtypeStruct((N_NODES, DQ), jnp.float32)
_sc_scatter = functools.partial(
    pl.kernel,
    out_type=(_quarter, _quarter),
    mesh=plsc.VectorSubcoreMesh(core_axis_name="c", subcore_axis_name="s",
                                num_cores=NC, num_subcores=NS),
    scratch_types=[
        pltpu.VMEM((HALF, CHUNK), jnp.int32),
        pltpu.VMEM((HALF, CHUNK), jnp.int32),
        pltpu.VMEM((CHUNK, DQ), jnp.float32),
        pltpu.VMEM((CHUNK, DQ), jnp.float32),
        pltpu.VMEM_SHARED((N_NODES, DQ), jnp.float32),
        pltpu.SemaphoreType.DMA,
        pltpu.SemaphoreType.DMA,
    ],
)(_sc_body)


def _mlp_body(x_ref, a0_ref, a1_ref,
              w1_ref, b1_ref, w2_ref, b2_ref, o_ref):
    h = x_ref[...] + jnp.concatenate([a0_ref[...], a1_ref[...]], axis=1)
    h = jnp.dot(h, w1_ref[...], preferred_element_type=jnp.float32) + b1_ref[...]
    h = jnp.maximum(h, 0.0)
    o_ref[...] = (jnp.dot(h, w2_ref[...], preferred_element_type=jnp.float32)
                  + b2_ref[...])


BLK = 1000


def _mlp(x, aggs, w1, b1, w2, b2):
    return pl.pallas_call(
        _mlp_body,
        grid=(N_NODES // BLK,),
        in_specs=[
            pl.BlockSpec((BLK, D), lambda i: (i, 0)),
            pl.BlockSpec((BLK, DQ), lambda i: (i, 0)),
            pl.BlockSpec((BLK, DQ), lambda i: (i, 0)),
            pl.BlockSpec((D, D), lambda i: (0, 0)),
            pl.BlockSpec((1, D), lambda i: (0, 0)),
            pl.BlockSpec((D, D), lambda i: (0, 0)),
            pl.BlockSpec((1, D), lambda i: (0, 0)),
        ],
        out_specs=pl.BlockSpec((BLK, D), lambda i: (i, 0)),
        out_shape=jax.ShapeDtypeStruct((N_NODES, D), jnp.float32),
    )(x, *aggs, w1, b1.reshape(1, D), w2, b2.reshape(1, D))


def kernel(x, edge_index, W1, b1, W2, b2):
    ei = edge_index.astype(jnp.int32).reshape(2, NS, NCHUNK, CHUNK)
    zeros = jnp.zeros((ROWS_PER_TILE, DQ), jnp.float32)
    aggs = _sc_scatter(x, ei[0], ei[1], zeros)
    return _mlp(x, aggs, W1, b1, W2, b2)
